# baseline (speedup 1.0000x reference)
import functools

import jax
import jax.numpy as jnp
from jax.experimental import pallas as pl
from jax.experimental.pallas import tpu as pltpu

DIM = 512
V = 50000
T = 64
K = 12
BV = 2048
NB = (V + BV - 1) // BV
VPAD = NB * BV
NEG = -3.0e38
HIGH = jax.lax.Precision.HIGHEST


def _combine_body(idx_ref, w_ref, s_ref, v_ref, p_ref, self_ref, l0_ref,
                  l1_ref, out_ref, acc_ref, *, n, finalize):
    i = pl.program_id(0)

    @pl.when(i == 0)
    def _():
        acc_ref[...] = jnp.zeros_like(acc_ref)

    lane = jax.lax.broadcasted_iota(jnp.int32, (1, 128), 1)
    w = jnp.sum(jnp.where(lane == i, w_ref[...], 0.0))
    row = s_ref[0] + v_ref[0] + p_ref[0]
    acc_ref[...] += w * row

    @pl.when(i == n - 1)
    def _():
        vec = acc_ref[...]
        if finalize:
            vec = jax.lax.dot_general(vec, l0_ref[...],
                                      (((1,), (0,)), ((), ())), precision=HIGH)
            vec = jax.lax.dot_general(vec, l1_ref[...],
                                      (((1,), (0,)), ((), ())), precision=HIGH)
            se = self_ref[...]
            nrm = jnp.sqrt(jnp.sum(se * se))
            s = jnp.where(nrm > 1e-8, se / nrm, se)
            n2 = jnp.sqrt(jnp.sum(s * s))
            s = jnp.where(n2 > 0.8, s * (0.8 / jnp.maximum(n2, 1e-8)), s)
            vec = vec - s
        out_ref[...] = vec


def _combine(idx, w, sym3, vis3, phys3, self2, l0, l1, n, finalize,
             interpret=False):
    body = functools.partial(_combine_body, n=n, finalize=finalize)
    grid_spec = pltpu.PrefetchScalarGridSpec(
        num_scalar_prefetch=1,
        grid=(n,),
        in_specs=[
            pl.BlockSpec((1, 128), lambda i, idx_ref: (0, 0)),
            pl.BlockSpec((1, 1, DIM), lambda i, idx_ref: (idx_ref[i], 0, 0)),
            pl.BlockSpec((1, 1, DIM), lambda i, idx_ref: (idx_ref[i], 0, 0)),
            pl.BlockSpec((1, 1, DIM), lambda i, idx_ref: (idx_ref[i], 0, 0)),
            pl.BlockSpec((1, DIM), lambda i, idx_ref: (0, 0)),
            pl.BlockSpec((DIM, DIM), lambda i, idx_ref: (0, 0)),
            pl.BlockSpec((DIM, DIM), lambda i, idx_ref: (0, 0)),
        ],
        out_specs=pl.BlockSpec((1, DIM), lambda i, idx_ref: (0, 0)),
        scratch_shapes=[pltpu.VMEM((1, DIM), jnp.float32)],
    )
    return pl.pallas_call(
        body,
        grid_spec=grid_spec,
        out_shape=jax.ShapeDtypeStruct((1, DIM), jnp.float32),
        interpret=interpret,
    )(idx, w, sym3, vis3, phys3, self2, l0, l1)


def _pass1_body(vec_ref, s_ref, v_ref, p_ref, w_ref, sim_ref):
    i = pl.program_id(0)
    wblk = 0.5 * s_ref[...] + 0.3 * v_ref[...] + 0.2 * p_ref[...]
    w_ref[...] = wblk
    sim = jax.lax.dot_general(vec_ref[...], wblk, (((1,), (1,)), ((), ())),
                              precision=HIGH)
    gid = i * BV + jax.lax.broadcasted_iota(jnp.int32, (1, BV), 1)
    sim_ref[0] = jnp.where(gid < V, sim, NEG)


def _pass1(vec, sym, vis, phys, interpret=False):
    return pl.pallas_call(
        _pass1_body,
        grid=(NB,),
        in_specs=[
            pl.BlockSpec((1, DIM), lambda i: (0, 0)),
            pl.BlockSpec((BV, DIM), lambda i: (i, 0)),
            pl.BlockSpec((BV, DIM), lambda i: (i, 0)),
            pl.BlockSpec((BV, DIM), lambda i: (i, 0)),
        ],
        out_specs=[
            pl.BlockSpec((BV, DIM), lambda i: (i, 0)),
            pl.BlockSpec((1, 1, BV), lambda i: (i, 0, 0)),
        ],
        out_shape=[
            jax.ShapeDtypeStruct((VPAD, DIM), jnp.float32),
            jax.ShapeDtypeStruct((NB, 1, BV), jnp.float32),
        ],
        interpret=interpret,
    )(vec, sym, vis, phys)


def _pass2_body(vec_ref, w_ref, sim_ref):
    i = pl.program_id(0)
    sim = jax.lax.dot_general(vec_ref[...], w_ref[...],
                              (((1,), (1,)), ((), ())), precision=HIGH)
    gid = i * BV + jax.lax.broadcasted_iota(jnp.int32, (1, BV), 1)
    sim_ref[0] = jnp.where(gid < V, sim, NEG)


def _pass2(vec, w, interpret=False):
    return pl.pallas_call(
        _pass2_body,
        grid=(NB,),
        in_specs=[
            pl.BlockSpec((1, DIM), lambda i: (0, 0)),
            pl.BlockSpec((BV, DIM), lambda i: (i, 0)),
        ],
        out_specs=pl.BlockSpec((1, 1, BV), lambda i: (i, 0, 0)),
        out_shape=jax.ShapeDtypeStruct((NB, 1, BV), jnp.float32),
        interpret=interpret,
    )(vec, w)


def _topk_body(sim_ref, idx_ref, w_ref, buf_ref):
    buf_ref[...] = sim_ref[...]
    gid = (jax.lax.broadcasted_iota(jnp.int32, (NB, 1, BV), 0) * BV
           + jax.lax.broadcasted_iota(jnp.int32, (NB, 1, BV), 2))
    lane = jax.lax.broadcasted_iota(jnp.int32, (1, 128), 1)

    def step(j, carry):
        vals, idxs = carry
        s = buf_ref[...]
        m = jnp.max(s)
        am = jnp.min(jnp.where(s == m, gid, jnp.int32(2**31 - 1)))
        buf_ref[...] = jnp.where(gid == am, NEG, s)
        vals = jnp.where(lane == j, m, vals)
        idxs = jnp.where(lane == j, am, idxs)
        return vals, idxs

    vals0 = jnp.zeros((1, 128), jnp.float32)
    idxs0 = jnp.zeros((1, 128), jnp.int32)
    vals, idxs = jax.lax.fori_loop(0, K, step, (vals0, idxs0))
    idx_ref[...] = idxs
    w_ref[...] = vals * (1.0 / 3.0)


def _topk(sim, interpret=False):
    return pl.pallas_call(
        _topk_body,
        grid=(1,),
        in_specs=[pl.BlockSpec((NB, 1, BV), lambda i: (0, 0, 0))],
        out_specs=[
            pl.BlockSpec((1, 128), lambda i: (0, 0)),
            pl.BlockSpec((1, 128), lambda i: (0, 0)),
        ],
        out_shape=[
            jax.ShapeDtypeStruct((1, 128), jnp.int32),
            jax.ShapeDtypeStruct((1, 128), jnp.float32),
        ],
        scratch_shapes=[pltpu.VMEM((NB, 1, BV), jnp.float32)],
        interpret=interpret,
    )(sim)


def _impl(tokens, sym_emb, vis_emb, phys_emb, self_emb, level0, level1,
          interpret=False):
    tokens = tokens.astype(jnp.int32)
    sym3 = sym_emb.reshape(V, 1, DIM)
    vis3 = vis_emb.reshape(V, 1, DIM)
    phys3 = phys_emb.reshape(V, 1, DIM)
    self2 = self_emb.reshape(1, DIM)

    enc_w = jnp.full((1, 128), 1.0 / (3.0 * T), jnp.float32)
    vec0 = _combine(tokens, enc_w, sym3, vis3, phys3, self2, level0, level1,
                    T, False, interpret)

    w_tab, sim1 = _pass1(vec0, sym_emb, vis_emb, phys_emb, interpret)
    idx1, wv1 = _topk(sim1, interpret)
    vec1 = _combine(idx1.reshape(128), wv1, sym3, vis3, phys3, self2,
                    level0, level1, K, False, interpret)

    sim2 = _pass2(vec1, w_tab, interpret)
    idx2, wv2 = _topk(sim2, interpret)
    ctx = _combine(idx2.reshape(128), wv2, sym3, vis3, phys3, self2,
                   level0, level1, K, True, interpret)

    score = _pass2(ctx, w_tab, interpret)
    return score.reshape(VPAD)[:V]


def kernel(tokens, sym_emb, vis_emb, phys_emb, self_emb, level0, level1):
    return _impl(tokens, sym_emb, vis_emb, phys_emb, self_emb, level0,
                 level1)


# speedup vs baseline: 3.9317x; 3.9317x over previous
import functools

import jax
import jax.numpy as jnp
from jax.experimental import pallas as pl
from jax.experimental.pallas import tpu as pltpu

DIM = 512
V = 50000
T = 64
K = 12
BV = 2048
NB = (V + BV - 1) // BV
VPAD = NB * BV
NEG = -3.0e38
HIGH = jax.lax.Precision.HIGHEST


def _combine_body(idx_ref, w_ref, s_hbm, v_hbm, p_hbm, self_ref, l0_ref,
                  l1_ref, out_ref, rs, rv, rp, sem, *, n, finalize):
    rs[...] = jnp.zeros_like(rs)
    rv[...] = jnp.zeros_like(rv)
    rp[...] = jnp.zeros_like(rp)
    cps = []
    for j in range(n):
        ij = idx_ref[j]
        for hbm, dst in ((s_hbm, rs), (v_hbm, rv), (p_hbm, rp)):
            cp = pltpu.make_async_copy(hbm.at[pl.ds(ij, 1), :],
                                       dst.at[pl.ds(j, 1), :], sem)
            cp.start()
            cps.append(cp)
    for cp in cps:
        cp.wait()
    rows = rs[...] + rv[...] + rp[...]
    vec = jax.lax.dot_general(w_ref[...], rows, (((1,), (0,)), ((), ())),
                              precision=HIGH)
    if finalize:
        vec = jax.lax.dot_general(vec, l0_ref[...],
                                  (((1,), (0,)), ((), ())), precision=HIGH)
        vec = jax.lax.dot_general(vec, l1_ref[...],
                                  (((1,), (0,)), ((), ())), precision=HIGH)
        se = self_ref[...]
        nrm = jnp.sqrt(jnp.sum(se * se))
        s = jnp.where(nrm > 1e-8, se / nrm, se)
        n2 = jnp.sqrt(jnp.sum(s * s))
        s = jnp.where(n2 > 0.8, s * (0.8 / jnp.maximum(n2, 1e-8)), s)
        vec = vec - s
    out_ref[...] = vec


def _combine(idx, w, sym, vis, phys, self2, l0, l1, n, finalize,
             interpret=False):
    body = functools.partial(_combine_body, n=n, finalize=finalize)
    grid_spec = pltpu.PrefetchScalarGridSpec(
        num_scalar_prefetch=1,
        grid=(1,),
        in_specs=[
            pl.BlockSpec((1, 128), lambda i, idx_ref: (0, 0)),
            pl.BlockSpec(memory_space=pl.ANY),
            pl.BlockSpec(memory_space=pl.ANY),
            pl.BlockSpec(memory_space=pl.ANY),
            pl.BlockSpec((1, DIM), lambda i, idx_ref: (0, 0)),
            pl.BlockSpec((DIM, DIM), lambda i, idx_ref: (0, 0)),
            pl.BlockSpec((DIM, DIM), lambda i, idx_ref: (0, 0)),
        ],
        out_specs=pl.BlockSpec((1, DIM), lambda i, idx_ref: (0, 0)),
        scratch_shapes=[
            pltpu.VMEM((128, DIM), jnp.float32),
            pltpu.VMEM((128, DIM), jnp.float32),
            pltpu.VMEM((128, DIM), jnp.float32),
            pltpu.SemaphoreType.DMA,
        ],
    )
    return pl.pallas_call(
        body,
        grid_spec=grid_spec,
        out_shape=jax.ShapeDtypeStruct((1, DIM), jnp.float32),
        interpret=interpret,
    )(idx, w, sym, vis, phys, self2, l0, l1)


def _pass1_body(vec_ref, s_ref, v_ref, p_ref, w_ref, sim_ref):
    i = pl.program_id(0)
    wblk = 0.5 * s_ref[...] + 0.3 * v_ref[...] + 0.2 * p_ref[...]
    w_ref[...] = wblk
    sim = jax.lax.dot_general(vec_ref[...], wblk, (((1,), (1,)), ((), ())),
                              precision=HIGH)
    gid = i * BV + jax.lax.broadcasted_iota(jnp.int32, (1, BV), 1)
    sim_ref[0] = jnp.where(gid < V, sim, NEG)


def _pass1(vec, sym, vis, phys, interpret=False):
    return pl.pallas_call(
        _pass1_body,
        grid=(NB,),
        in_specs=[
            pl.BlockSpec((1, DIM), lambda i: (0, 0)),
            pl.BlockSpec((BV, DIM), lambda i: (i, 0)),
            pl.BlockSpec((BV, DIM), lambda i: (i, 0)),
            pl.BlockSpec((BV, DIM), lambda i: (i, 0)),
        ],
        out_specs=[
            pl.BlockSpec((BV, DIM), lambda i: (i, 0)),
            pl.BlockSpec((1, 1, BV), lambda i: (i, 0, 0)),
        ],
        out_shape=[
            jax.ShapeDtypeStruct((VPAD, DIM), jnp.float32),
            jax.ShapeDtypeStruct((NB, 1, BV), jnp.float32),
        ],
        interpret=interpret,
    )(vec, sym, vis, phys)


def _pass2_body(vec_ref, w_ref, sim_ref):
    i = pl.program_id(0)
    sim = jax.lax.dot_general(vec_ref[...], w_ref[...],
                              (((1,), (1,)), ((), ())), precision=HIGH)
    gid = i * BV + jax.lax.broadcasted_iota(jnp.int32, (1, BV), 1)
    sim_ref[0] = jnp.where(gid < V, sim, NEG)


def _pass2(vec, w, interpret=False):
    return pl.pallas_call(
        _pass2_body,
        grid=(NB,),
        in_specs=[
            pl.BlockSpec((1, DIM), lambda i: (0, 0)),
            pl.BlockSpec((BV, DIM), lambda i: (i, 0)),
        ],
        out_specs=pl.BlockSpec((1, 1, BV), lambda i: (i, 0, 0)),
        out_shape=jax.ShapeDtypeStruct((NB, 1, BV), jnp.float32),
        interpret=interpret,
    )(vec, w)


def _topk_body(sim_ref, idx_ref, w_ref, buf_ref):
    buf_ref[...] = sim_ref[...]
    gid = (jax.lax.broadcasted_iota(jnp.int32, (NB, 1, BV), 0) * BV
           + jax.lax.broadcasted_iota(jnp.int32, (NB, 1, BV), 2))
    lane = jax.lax.broadcasted_iota(jnp.int32, (1, 128), 1)

    def step(j, carry):
        vals, idxs = carry
        s = buf_ref[...]
        m = jnp.max(s)
        am = jnp.min(jnp.where(s == m, gid, jnp.int32(2**31 - 1)))
        buf_ref[...] = jnp.where(gid == am, NEG, s)
        vals = jnp.where(lane == j, m, vals)
        idxs = jnp.where(lane == j, am, idxs)
        return vals, idxs

    vals0 = jnp.zeros((1, 128), jnp.float32)
    idxs0 = jnp.zeros((1, 128), jnp.int32)
    vals, idxs = jax.lax.fori_loop(0, K, step, (vals0, idxs0))
    idx_ref[...] = idxs
    w_ref[...] = vals * (1.0 / 3.0)


def _topk(sim, interpret=False):
    return pl.pallas_call(
        _topk_body,
        grid=(1,),
        in_specs=[pl.BlockSpec((NB, 1, BV), lambda i: (0, 0, 0))],
        out_specs=[
            pl.BlockSpec((1, 128), lambda i: (0, 0)),
            pl.BlockSpec((1, 128), lambda i: (0, 0)),
        ],
        out_shape=[
            jax.ShapeDtypeStruct((1, 128), jnp.int32),
            jax.ShapeDtypeStruct((1, 128), jnp.float32),
        ],
        scratch_shapes=[pltpu.VMEM((NB, 1, BV), jnp.float32)],
        interpret=interpret,
    )(sim)


def _impl(tokens, sym_emb, vis_emb, phys_emb, self_emb, level0, level1,
          interpret=False):
    tokens = tokens.astype(jnp.int32)
    self2 = self_emb.reshape(1, DIM)

    lane = jnp.arange(128, dtype=jnp.int32)[None, :]
    enc_w = jnp.where(lane < T, 1.0 / (3.0 * T), 0.0).astype(jnp.float32)
    tok_pad = jnp.concatenate([tokens, jnp.zeros(128 - T, jnp.int32)])
    vec0 = _combine(tok_pad, enc_w, sym_emb, vis_emb, phys_emb, self2,
                    level0, level1, T, False, interpret)

    w_tab, sim1 = _pass1(vec0, sym_emb, vis_emb, phys_emb, interpret)
    idx1, wv1 = _topk(sim1, interpret)
    vec1 = _combine(idx1.reshape(128), wv1, sym_emb, vis_emb, phys_emb,
                    self2, level0, level1, K, False, interpret)

    sim2 = _pass2(vec1, w_tab, interpret)
    idx2, wv2 = _topk(sim2, interpret)
    ctx = _combine(idx2.reshape(128), wv2, sym_emb, vis_emb, phys_emb,
                   self2, level0, level1, K, True, interpret)

    score = _pass2(ctx, w_tab, interpret)
    return score.reshape(VPAD)[:V]


def kernel(tokens, sym_emb, vis_emb, phys_emb, self_emb, level0, level1):
    return _impl(tokens, sym_emb, vis_emb, phys_emb, self_emb, level0,
                 level1)


# speedup vs baseline: 4.3220x; 1.0993x over previous
import functools

import jax
import jax.numpy as jnp
from jax.experimental import pallas as pl
from jax.experimental.pallas import tpu as pltpu

DIM = 512
V = 50000
T = 64
K = 12
BV = 2048
NB = (V + BV - 1) // BV
VPAD = NB * BV
NEG = -3.0e38
HIGH = jax.lax.Precision.HIGHEST
F32 = jnp.float32
BF16 = jnp.bfloat16
_DN_T = (((1,), (1,)), ((), ()))
_DN_N = (((1,), (0,)), ((), ()))


def _combine_body(idx_ref, w_ref, s_hbm, v_hbm, p_hbm, self_ref, l0_ref,
                  l1_ref, out_ref, rs, rv, rp, sem, *, n, mode, finalize):
    rs[...] = jnp.zeros_like(rs)
    rv[...] = jnp.zeros_like(rv)
    rp[...] = jnp.zeros_like(rp)
    cps = []
    for j in range(n):
        ij = idx_ref[j]
        for hbm, dst in ((s_hbm, rs), (v_hbm, rv), (p_hbm, rp)):
            cp = pltpu.make_async_copy(hbm.at[pl.ds(ij, 1), :],
                                       dst.at[pl.ds(j, 1), :], sem)
            cp.start()
            cps.append(cp)
    for cp in cps:
        cp.wait()
    if mode == "encode":
        rows = rs[...] + rv[...] + rp[...]
        vec = jax.lax.dot_general(w_ref[...], rows, _DN_N, precision=HIGH)
    else:
        wb = w_ref[...].astype(BF16)
        d1 = jax.lax.dot_general(wb, rs[...].astype(BF16), _DN_N,
                                 preferred_element_type=F32)
        d2 = jax.lax.dot_general(wb, rv[...].astype(BF16), _DN_N,
                                 preferred_element_type=F32)
        d3 = jax.lax.dot_general(wb, rp[...].astype(BF16), _DN_N,
                                 preferred_element_type=F32)
        vec = (d1 + d2 + d3) * (1.0 / 3.0)
    if finalize:
        vec = jax.lax.dot_general(vec, l0_ref[...], _DN_N)
        vec = jax.lax.dot_general(vec, l1_ref[...], _DN_N)
        se = self_ref[...]
        nrm = jnp.sqrt(jnp.sum(se * se))
        s = jnp.where(nrm > 1e-8, se / nrm, se)
        n2 = jnp.sqrt(jnp.sum(s * s))
        s = jnp.where(n2 > 0.8, s * (0.8 / jnp.maximum(n2, 1e-8)), s)
        vec = vec - s
    out_ref[...] = vec


def _combine(idx, w, sym, vis, phys, self2, l0, l1, n, mode, finalize,
             interpret=False):
    body = functools.partial(_combine_body, n=n, mode=mode,
                             finalize=finalize)
    grid_spec = pltpu.PrefetchScalarGridSpec(
        num_scalar_prefetch=1,
        grid=(1,),
        in_specs=[
            pl.BlockSpec((1, 128), lambda i, idx_ref: (0, 0)),
            pl.BlockSpec(memory_space=pl.ANY),
            pl.BlockSpec(memory_space=pl.ANY),
            pl.BlockSpec(memory_space=pl.ANY),
            pl.BlockSpec((1, DIM), lambda i, idx_ref: (0, 0)),
            pl.BlockSpec((DIM, DIM), lambda i, idx_ref: (0, 0)),
            pl.BlockSpec((DIM, DIM), lambda i, idx_ref: (0, 0)),
        ],
        out_specs=pl.BlockSpec((1, DIM), lambda i, idx_ref: (0, 0)),
        scratch_shapes=[
            pltpu.VMEM((128, DIM), jnp.float32),
            pltpu.VMEM((128, DIM), jnp.float32),
            pltpu.VMEM((128, DIM), jnp.float32),
            pltpu.SemaphoreType.DMA,
        ],
    )
    return pl.pallas_call(
        body,
        grid_spec=grid_spec,
        out_shape=jax.ShapeDtypeStruct((1, DIM), jnp.float32),
        interpret=interpret,
    )(idx, w, sym, vis, phys, self2, l0, l1)


def _pass1_body(vec_ref, s_ref, v_ref, p_ref, sb_ref, vb_ref, pb_ref,
                sim_ref):
    i = pl.program_id(0)
    vec = vec_ref[...]
    s = s_ref[...]
    v = v_ref[...]
    p = p_ref[...]
    sb_ref[...] = s.astype(BF16)
    vb_ref[...] = v.astype(BF16)
    pb_ref[...] = p.astype(BF16)
    d1 = jax.lax.dot_general(vec, s, _DN_T)
    d2 = jax.lax.dot_general(vec, v, _DN_T)
    d3 = jax.lax.dot_general(vec, p, _DN_T)
    sim = d1 * 0.5 + d2 * 0.3 + d3 * 0.2
    gid = i * BV + jax.lax.broadcasted_iota(jnp.int32, (1, BV), 1)
    sim_ref[0] = jnp.where(gid < V, sim, NEG)


def _pass1(vec, sym, vis, phys, interpret=False):
    return pl.pallas_call(
        _pass1_body,
        grid=(NB,),
        in_specs=[
            pl.BlockSpec((1, DIM), lambda i: (0, 0)),
            pl.BlockSpec((BV, DIM), lambda i: (i, 0)),
            pl.BlockSpec((BV, DIM), lambda i: (i, 0)),
            pl.BlockSpec((BV, DIM), lambda i: (i, 0)),
        ],
        out_specs=[
            pl.BlockSpec((BV, DIM), lambda i: (i, 0)),
            pl.BlockSpec((BV, DIM), lambda i: (i, 0)),
            pl.BlockSpec((BV, DIM), lambda i: (i, 0)),
            pl.BlockSpec((1, 1, BV), lambda i: (i, 0, 0)),
        ],
        out_shape=[
            jax.ShapeDtypeStruct((VPAD, DIM), BF16),
            jax.ShapeDtypeStruct((VPAD, DIM), BF16),
            jax.ShapeDtypeStruct((VPAD, DIM), BF16),
            jax.ShapeDtypeStruct((NB, 1, BV), jnp.float32),
        ],
        interpret=interpret,
    )(vec, sym, vis, phys)


def _pass2_body(vec_ref, sb_ref, vb_ref, pb_ref, sim_ref):
    i = pl.program_id(0)
    vec = vec_ref[...].astype(BF16)
    d1 = jax.lax.dot_general(vec, sb_ref[...], _DN_T,
                             preferred_element_type=F32)
    d2 = jax.lax.dot_general(vec, vb_ref[...], _DN_T,
                             preferred_element_type=F32)
    d3 = jax.lax.dot_general(vec, pb_ref[...], _DN_T,
                             preferred_element_type=F32)
    sim = d1 * 0.5 + d2 * 0.3 + d3 * 0.2
    gid = i * BV + jax.lax.broadcasted_iota(jnp.int32, (1, BV), 1)
    sim_ref[0] = jnp.where(gid < V, sim, NEG)


def _pass2(vec, sb, vb, pb, interpret=False):
    return pl.pallas_call(
        _pass2_body,
        grid=(NB,),
        in_specs=[
            pl.BlockSpec((1, DIM), lambda i: (0, 0)),
            pl.BlockSpec((BV, DIM), lambda i: (i, 0)),
            pl.BlockSpec((BV, DIM), lambda i: (i, 0)),
            pl.BlockSpec((BV, DIM), lambda i: (i, 0)),
        ],
        out_specs=pl.BlockSpec((1, 1, BV), lambda i: (i, 0, 0)),
        out_shape=jax.ShapeDtypeStruct((NB, 1, BV), jnp.float32),
        interpret=interpret,
    )(vec, sb, vb, pb)


def _topk_body(sim_ref, idx_ref, w_ref, buf_ref):
    buf_ref[...] = sim_ref[...]
    gid = (jax.lax.broadcasted_iota(jnp.int32, (NB, 1, BV), 0) * BV
           + jax.lax.broadcasted_iota(jnp.int32, (NB, 1, BV), 2))
    lane = jax.lax.broadcasted_iota(jnp.int32, (1, 128), 1)

    def step(j, carry):
        vals, idxs = carry
        s = buf_ref[...]
        m = jnp.max(s)
        am = jnp.min(jnp.where(s == m, gid, jnp.int32(2**31 - 1)))
        buf_ref[...] = jnp.where(gid == am, NEG, s)
        vals = jnp.where(lane == j, m, vals)
        idxs = jnp.where(lane == j, am, idxs)
        return vals, idxs

    vals0 = jnp.zeros((1, 128), jnp.float32)
    idxs0 = jnp.zeros((1, 128), jnp.int32)
    vals, idxs = jax.lax.fori_loop(0, K, step, (vals0, idxs0))
    idx_ref[...] = idxs
    w_ref[...] = vals


def _topk(sim, interpret=False):
    return pl.pallas_call(
        _topk_body,
        grid=(1,),
        in_specs=[pl.BlockSpec((NB, 1, BV), lambda i: (0, 0, 0))],
        out_specs=[
            pl.BlockSpec((1, 128), lambda i: (0, 0)),
            pl.BlockSpec((1, 128), lambda i: (0, 0)),
        ],
        out_shape=[
            jax.ShapeDtypeStruct((1, 128), jnp.int32),
            jax.ShapeDtypeStruct((1, 128), jnp.float32),
        ],
        scratch_shapes=[pltpu.VMEM((NB, 1, BV), jnp.float32)],
        interpret=interpret,
    )(sim)


NW = 32
CHUNK = VPAD // NW
NVR = CHUNK // 16
IMAX = 2**31 - 1


def _sc_localtopk_body(sim_hbm, valout_hbm, idxout_hbm, chunk, vstage,
                       istage, sem):
    wid = jax.lax.axis_index("s") * 2 + jax.lax.axis_index("c")
    base = wid * CHUNK
    pltpu.sync_copy(sim_hbm.at[pl.ds(base, CHUNK)], chunk)

    lane16 = jax.lax.iota(jnp.int32, 16)
    negv = jnp.full((16,), NEG, jnp.float32)
    imaxv = jnp.full((16,), IMAX, jnp.int32)
    vals16 = negv
    idxs16 = imaxv

    gd = jax.lax.GatherDimensionNumbers(
        offset_dims=(), collapsed_slice_dims=(0,), start_index_map=(0,))

    def rot(x, sh):
        idx = (((lane16 + sh) & 15))[:, None]
        return jax.lax.gather(
            x, idx, gd, slice_sizes=(1,),
            mode=jax.lax.GatherScatterMode.PROMISE_IN_BOUNDS)

    def tree_max(x):
        for sh in (8, 4, 2, 1):
            x = jnp.maximum(x, rot(x, sh))
        return x

    def tree_min(x):
        for sh in (8, 4, 2, 1):
            x = jnp.minimum(x, rot(x, sh))
        return x

    pv = jnp.full((16,), 3.0e38, jnp.float32)
    pi = jnp.full((16,), -1, jnp.int32)
    for j in range(K):
        def scan_body(i, carry):
            m, am = carry
            v = chunk[pl.ds(i * 16, 16)]
            idx = i * 16 + lane16
            elig = (v < pv) | ((v == pv) & (idx > pi))
            upd = elig & (v > m)
            return jnp.where(upd, v, m), jnp.where(upd, idx, am)

        m, am = jax.lax.fori_loop(0, NVR, scan_body,
                                  (negv, jnp.zeros((16,), jnp.int32)))
        mxv = tree_max(m)
        amv = tree_min(jnp.where(m == mxv, am, imaxv))
        vals16 = jnp.where(lane16 == j, mxv, vals16)
        idxs16 = jnp.where(lane16 == j, amv + base, idxs16)
        pv, pi = mxv, amv

    vstage[pl.ds(0, 16)] = vals16
    istage[pl.ds(0, 16)] = idxs16
    for s in range(1, 8):
        vstage[pl.ds(s * 16, 16)] = negv
        istage[pl.ds(s * 16, 16)] = imaxv
    pltpu.sync_copy(vstage, valout_hbm.at[wid])
    pltpu.sync_copy(istage, idxout_hbm.at[wid])


def _sc_localtopk(sim_flat):
    from jax.experimental.pallas import tpu_sc as plsc
    mesh = plsc.VectorSubcoreMesh(core_axis_name="c", subcore_axis_name="s")
    fn = functools.partial(
        pl.kernel,
        mesh=mesh,
        out_type=[
            jax.ShapeDtypeStruct((NW, 128), jnp.float32),
            jax.ShapeDtypeStruct((NW, 128), jnp.int32),
        ],
        scratch_types=[
            pltpu.VMEM((CHUNK,), jnp.float32),
            pltpu.VMEM((128,), jnp.float32),
            pltpu.VMEM((128,), jnp.int32),
            pltpu.SemaphoreType.DMA,
        ],
    )(_sc_localtopk_body)
    return fn(sim_flat)


def _merge_body(val_ref, idx_ref, oidx_ref, ow_ref):
    lane = jax.lax.broadcasted_iota(jnp.int32, (1, 128), 1)

    def step(j, carry):
        vals, idxs, cv, ci = carry
        m = jnp.max(cv)
        am = jnp.min(jnp.where(cv == m, ci, IMAX))
        cv = jnp.where(ci == am, NEG, cv)
        vals = jnp.where(lane == j, m, vals)
        idxs = jnp.where(lane == j, am, idxs)
        return vals, idxs, cv, ci

    vals0 = jnp.zeros((1, 128), jnp.float32)
    idxs0 = jnp.zeros((1, 128), jnp.int32)
    vals, idxs, _, _ = jax.lax.fori_loop(
        0, K, step, (vals0, idxs0, val_ref[...], idx_ref[...]))
    oidx_ref[...] = idxs
    ow_ref[...] = vals


def _merge_topk(cands_v, cands_i, interpret=False):
    return pl.pallas_call(
        _merge_body,
        grid=(1,),
        in_specs=[
            pl.BlockSpec((NW, 128), lambda i: (0, 0)),
            pl.BlockSpec((NW, 128), lambda i: (0, 0)),
        ],
        out_specs=[
            pl.BlockSpec((1, 128), lambda i: (0, 0)),
            pl.BlockSpec((1, 128), lambda i: (0, 0)),
        ],
        out_shape=[
            jax.ShapeDtypeStruct((1, 128), jnp.int32),
            jax.ShapeDtypeStruct((1, 128), jnp.float32),
        ],
        interpret=interpret,
    )(cands_v, cands_i)


def _impl(tokens, sym_emb, vis_emb, phys_emb, self_emb, level0, level1,
          interpret=False):
    tokens = tokens.astype(jnp.int32)
    self2 = self_emb.reshape(1, DIM)

    lane = jnp.arange(128, dtype=jnp.int32)[None, :]
    enc_w = jnp.where(lane < T, 1.0 / (3.0 * T), 0.0).astype(jnp.float32)
    tok_pad = jnp.concatenate([tokens, jnp.zeros(128 - T, jnp.int32)])
    vec0 = _combine(tok_pad, enc_w, sym_emb, vis_emb, phys_emb, self2,
                    level0, level1, T, "encode", False, interpret)

    def select(sim):
        if interpret:
            return _topk(sim, interpret)
        cv, ci = _sc_localtopk(sim.reshape(VPAD))
        return _merge_topk(cv, ci)

    sb, vb, pb, sim1 = _pass1(vec0, sym_emb, vis_emb, phys_emb, interpret)
    idx1, wv1 = select(sim1)
    vec1 = _combine(idx1.reshape(128), wv1, sym_emb, vis_emb, phys_emb,
                    self2, level0, level1, K, "topk", False, interpret)

    sim2 = _pass2(vec1, sb, vb, pb, interpret)
    idx2, wv2 = select(sim2)
    ctx = _combine(idx2.reshape(128), wv2, sym_emb, vis_emb, phys_emb,
                   self2, level0, level1, K, "topk", True, interpret)

    score = _pass2(ctx, sb, vb, pb, interpret)
    return score.reshape(VPAD)[:V]


def kernel(tokens, sym_emb, vis_emb, phys_emb, self_emb, level0, level1):
    return _impl(tokens, sym_emb, vis_emb, phys_emb, self_emb, level0,
                 level1)
